# baseline (speedup 1.0000x reference)
import functools

import jax
import jax.numpy as jnp
from jax import lax
from jax.experimental import pallas as pl
from jax.experimental.pallas import tpu as pltpu
from jax.experimental.pallas import tpu_sc as plsc

N = 10000
M = 32
NORIG = 92
A = 128
BF = 16
C2 = 256
EPS = 1e-5
PAIRS = N * M

NB = 80
GRID_P = N // NB
RB = 1000
GRID_N = N // RB
CHUNK = 80
NW = 32
BPW = PAIRS // NW
NCHUNK = BPW // CHUNK


def _softplus(x):
    return jnp.maximum(x, 0.0) + jnp.log1p(jnp.exp(-jnp.abs(x)))


def _sigmoid(x):
    return 1.0 / (1.0 + jnp.exp(-x))


def _embed_body(x_ref, w0_ref, b0_ref, w1_ref, bfc_ref, emb_ref, p1_ref):
    x = x_ref[...]
    emb = jnp.dot(x, w0_ref[...], preferred_element_type=jnp.float32, precision=lax.Precision.HIGHEST) + b0_ref[...]
    emb_ref[...] = emb
    p1_ref[...] = (
        jnp.dot(emb, w1_ref[...], preferred_element_type=jnp.float32, precision=lax.Precision.HIGHEST) + bfc_ref[...]
    )


def _embed(xpad, w0pad, b0, w1, bfc):
    return pl.pallas_call(
        _embed_body,
        grid=(GRID_N,),
        in_specs=[
            pl.BlockSpec((RB, A), lambda i: (i, 0)),
            pl.BlockSpec((A, A), lambda i: (0, 0)),
            pl.BlockSpec((1, A), lambda i: (0, 0)),
            pl.BlockSpec((A, C2), lambda i: (0, 0)),
            pl.BlockSpec((1, C2), lambda i: (0, 0)),
        ],
        out_specs=[
            pl.BlockSpec((RB, A), lambda i: (i, 0)),
            pl.BlockSpec((RB, C2), lambda i: (i, 0)),
        ],
        out_shape=[
            jax.ShapeDtypeStruct((N, A), jnp.float32),
            jax.ShapeDtypeStruct((N, C2), jnp.float32),
        ],
    )(xpad, w0pad, b0, w1, bfc)


def _update_body(prev_ref, nbr_ref, sums2_ref, g2_ref, b2_ref, w1_ref, bfc_ref,
                 s_ref, p1_ref):
    mean = sums2_ref[0:1, :] / N
    var = sums2_ref[1:2, :] / N - mean * mean
    scale = g2_ref[...] * lax.rsqrt(var + EPS)
    shift = b2_ref[...] - mean * scale
    s = _softplus(prev_ref[...] + nbr_ref[...] * scale + shift)
    s_ref[...] = s
    p1_ref[...] = (
        jnp.dot(s, w1_ref[...], preferred_element_type=jnp.float32, precision=lax.Precision.HIGHEST) + bfc_ref[...]
    )


def _update(prev, nbr, sums2, g2, b2, w1, bfc):
    return pl.pallas_call(
        _update_body,
        grid=(GRID_N,),
        in_specs=[
            pl.BlockSpec((RB, A), lambda i: (i, 0)),
            pl.BlockSpec((RB, A), lambda i: (i, 0)),
            pl.BlockSpec((2, A), lambda i: (0, 0)),
            pl.BlockSpec((1, A), lambda i: (0, 0)),
            pl.BlockSpec((1, A), lambda i: (0, 0)),
            pl.BlockSpec((A, C2), lambda i: (0, 0)),
            pl.BlockSpec((1, C2), lambda i: (0, 0)),
        ],
        out_specs=[
            pl.BlockSpec((RB, A), lambda i: (i, 0)),
            pl.BlockSpec((RB, C2), lambda i: (i, 0)),
        ],
        out_shape=[
            jax.ShapeDtypeStruct((N, A), jnp.float32),
            jax.ShapeDtypeStruct((N, C2), jnp.float32),
        ],
    )(prev, nbr, sums2, g2, b2, w1, bfc)


def _gather_rows(table, idx_flat):
    info = plsc.get_sparse_core_info()
    nc = info.num_cores

    @functools.partial(
        pl.kernel,
        mesh=plsc.VectorSubcoreMesh(core_axis_name="c", subcore_axis_name="s"),
        out_type=jax.ShapeDtypeStruct((PAIRS, A), jnp.float32),
        scratch_types=[
            pltpu.VMEM((BPW,), jnp.int32),
            pltpu.VMEM((2, CHUNK, A), jnp.float32),
            pltpu.SemaphoreType.DMA,
            pltpu.SemaphoreType.DMA,
        ],
    )
    def k(table_hbm, idx_hbm, out_hbm, idx_v, rows_v, sem0, sem1):
        wid = lax.axis_index("s") * nc + lax.axis_index("c")
        base = pl.multiple_of(wid * BPW, 8)
        pltpu.sync_copy(idx_hbm.at[pl.ds(base, BPW)], idx_v)
        sems = (sem0, sem1)
        cps = [None, None]
        for b in range(2):
            cps[b] = pltpu.async_copy(
                table_hbm.at[idx_v.at[pl.ds(b * CHUNK, CHUNK)]],
                rows_v.at[b], sems[b])
        for kk in range(NCHUNK):
            b = kk % 2
            cps[b].wait()
            pltpu.sync_copy(rows_v.at[b],
                            out_hbm.at[pl.ds(base + kk * CHUNK, CHUNK)])
            nxt = kk + 2
            if nxt < NCHUNK:
                cps[b] = pltpu.async_copy(
                    table_hbm.at[idx_v.at[pl.ds(nxt * CHUNK, CHUNK)]],
                    rows_v.at[b], sems[b])

    return k(table, idx_flat)


def _stats_body(g_ref, bond_ref, p1_ref, w2_ref, w3_ref, out_ref, acc_ref):
    i = pl.program_id(0)

    @pl.when(i == 0)
    def _init():
        acc_ref[...] = jnp.zeros_like(acc_ref)

    g = g_ref[...].reshape(NB * M, A)
    bond = bond_ref[...].reshape(NB * M, BF)
    t = (jnp.dot(g, w2_ref[...], preferred_element_type=jnp.float32, precision=lax.Precision.HIGHEST)
         + jnp.dot(bond, w3_ref[...], preferred_element_type=jnp.float32, precision=lax.Precision.HIGHEST))
    t = (t.reshape(NB, M, C2) + p1_ref[...][:, None, :]).reshape(NB * M, C2)
    acc_ref[0:1, :] += jnp.sum(t, axis=0, keepdims=True)
    acc_ref[1:2, :] += jnp.sum(t * t, axis=0, keepdims=True)

    @pl.when(i == GRID_P - 1)
    def _fin():
        out_ref[...] = acc_ref[...]


def _stats(g3, bond3, p1, w2, w3):
    return pl.pallas_call(
        _stats_body,
        grid=(GRID_P,),
        in_specs=[
            pl.BlockSpec((NB, M, A), lambda i: (i, 0, 0)),
            pl.BlockSpec((NB, M, BF), lambda i: (i, 0, 0)),
            pl.BlockSpec((NB, C2), lambda i: (i, 0)),
            pl.BlockSpec((A, C2), lambda i: (0, 0)),
            pl.BlockSpec((BF, C2), lambda i: (0, 0)),
        ],
        out_specs=pl.BlockSpec((2, C2), lambda i: (0, 0)),
        out_shape=jax.ShapeDtypeStruct((2, C2), jnp.float32),
        scratch_shapes=[pltpu.VMEM((2, C2), jnp.float32)],
    )(g3, bond3, p1, w2, w3)


def _main_body(g_ref, bond_ref, p1_ref, w2_ref, w3_ref, sums_ref, g1_ref, b1_ref,
               nbr_ref, out2_ref, acc_ref):
    i = pl.program_id(0)

    @pl.when(i == 0)
    def _init():
        acc_ref[...] = jnp.zeros_like(acc_ref)

    mean = sums_ref[0:1, :] / PAIRS
    var = sums_ref[1:2, :] / PAIRS - mean * mean
    scale = g1_ref[...] * lax.rsqrt(var + EPS)
    shift = b1_ref[...] - mean * scale

    g = g_ref[...].reshape(NB * M, A)
    bond = bond_ref[...].reshape(NB * M, BF)
    t = (jnp.dot(g, w2_ref[...], preferred_element_type=jnp.float32, precision=lax.Precision.HIGHEST)
         + jnp.dot(bond, w3_ref[...], preferred_element_type=jnp.float32, precision=lax.Precision.HIGHEST))
    t = (t.reshape(NB, M, C2) + p1_ref[...][:, None, :]).reshape(NB * M, C2)
    t = t * scale + shift
    f = _sigmoid(t[:, :A])
    c = _softplus(t[:, A:])
    nbr = jnp.sum((f * c).reshape(NB, M, A), axis=1)
    nbr_ref[...] = nbr
    acc_ref[0:1, :] += jnp.sum(nbr, axis=0, keepdims=True)
    acc_ref[1:2, :] += jnp.sum(nbr * nbr, axis=0, keepdims=True)

    @pl.when(i == GRID_P - 1)
    def _fin():
        out2_ref[...] = acc_ref[...]


def _main(g3, bond3, p1, w2, w3, sums, g1, b1):
    return pl.pallas_call(
        _main_body,
        grid=(GRID_P,),
        in_specs=[
            pl.BlockSpec((NB, M, A), lambda i: (i, 0, 0)),
            pl.BlockSpec((NB, M, BF), lambda i: (i, 0, 0)),
            pl.BlockSpec((NB, C2), lambda i: (i, 0)),
            pl.BlockSpec((A, C2), lambda i: (0, 0)),
            pl.BlockSpec((BF, C2), lambda i: (0, 0)),
            pl.BlockSpec((2, C2), lambda i: (0, 0)),
            pl.BlockSpec((1, C2), lambda i: (0, 0)),
            pl.BlockSpec((1, C2), lambda i: (0, 0)),
        ],
        out_specs=[
            pl.BlockSpec((NB, A), lambda i: (i, 0)),
            pl.BlockSpec((2, A), lambda i: (0, 0)),
        ],
        out_shape=[
            jax.ShapeDtypeStruct((N, A), jnp.float32),
            jax.ShapeDtypeStruct((2, A), jnp.float32),
        ],
        scratch_shapes=[pltpu.VMEM((2, A), jnp.float32)],
    )(g3, bond3, p1, w2, w3, sums, g1, b1)


def _final_body(prev_ref, nbr_ref, sums2_ref, g2_ref, b2_ref, fap_ref,
                w2a_ref, w2b_ref, b2f_ref, w3f_ref, b3f_ref, out_ref, acc_ref):
    i = pl.program_id(0)

    @pl.when(i == 0)
    def _init():
        acc_ref[...] = jnp.zeros_like(acc_ref)

    mean = sums2_ref[0:1, :] / N
    var = sums2_ref[1:2, :] / N - mean * mean
    scale = g2_ref[...] * lax.rsqrt(var + EPS)
    shift = b2_ref[...] - mean * scale
    s = _softplus(prev_ref[...] + nbr_ref[...] * scale + shift)
    acc_ref[...] += jnp.sum(s, axis=0, keepdims=True)

    @pl.when(i == GRID_N - 1)
    def _fin():
        feat = acc_ref[...] / N
        h = (jnp.dot(_softplus(feat), w2a_ref[...],
                     preferred_element_type=jnp.float32, precision=lax.Precision.HIGHEST)
             + _softplus(fap_ref[...]) * w2b_ref[...] + b2f_ref[...])
        out_ref[...] = (jnp.dot(_softplus(h), w3f_ref[...],
                                preferred_element_type=jnp.float32, precision=lax.Precision.HIGHEST)
                        + b3f_ref[...])


def _final(prev, nbr, sums2, g2, b2, fap, w2a, w2b, b2f, w3f, b3f):
    return pl.pallas_call(
        _final_body,
        grid=(GRID_N,),
        in_specs=[
            pl.BlockSpec((RB, A), lambda i: (i, 0)),
            pl.BlockSpec((RB, A), lambda i: (i, 0)),
            pl.BlockSpec((2, A), lambda i: (0, 0)),
            pl.BlockSpec((1, A), lambda i: (0, 0)),
            pl.BlockSpec((1, A), lambda i: (0, 0)),
            pl.BlockSpec((1, 1), lambda i: (0, 0)),
            pl.BlockSpec((A, HID), lambda i: (0, 0)),
            pl.BlockSpec((1, HID), lambda i: (0, 0)),
            pl.BlockSpec((1, HID), lambda i: (0, 0)),
            pl.BlockSpec((HID, 1), lambda i: (0, 0)),
            pl.BlockSpec((1, 1), lambda i: (0, 0)),
        ],
        out_specs=pl.BlockSpec((1, 1), lambda i: (0, 0)),
        out_shape=jax.ShapeDtypeStruct((1, 1), jnp.float32),
        scratch_shapes=[pltpu.VMEM((1, A), jnp.float32)],
    )(prev, nbr, sums2, g2, b2, fap, w2a, w2b, b2f, w3f, b3f)


HID = 128


def kernel(site_features, bond_features, bond_indices, feature_after_pooing,
           fc1_W, fc1_b, conv_params, fc2_W, fc2_b, fc3_W, fc3_b):
    xpad = jnp.pad(site_features, ((0, 0), (0, A - NORIG)))
    w0pad = jnp.pad(fc1_W, ((0, A - NORIG), (0, 0)))
    idx = bond_indices.reshape(-1).astype(jnp.int32)
    bond3 = bond_features

    site = None
    nbr = sums2 = None
    for l in range(3):
        Wfc, bfc, g1, b1, g2p, b2p = conv_params[l]
        w1 = Wfc[:A]
        w2 = Wfc[A:2 * A]
        w3 = Wfc[2 * A:]
        if l == 0:
            site, p1 = _embed(xpad, w0pad, fc1_b[None, :], w1, bfc[None, :])
        else:
            pWfc, pbfc, pg1, pb1, pg2, pb2 = conv_params[l - 1]
            site, p1 = _update(site, nbr, sums2, pg2[None, :], pb2[None, :],
                               w1, bfc[None, :])
        g3 = _gather_rows(site, idx).reshape(N, M, A)
        sums = _stats(g3, bond3, p1, w2, w3)
        nbr, sums2 = _main(g3, bond3, p1, w2, w3, sums,
                           g1[None, :], b1[None, :])

    _, _, _, _, g2l, b2l = conv_params[2]
    pred = _final(site, nbr, sums2, g2l[None, :], b2l[None, :],
                  feature_after_pooing[:, None],
                  fc2_W[:A], fc2_W[A:A + 1], fc2_b[None, :], fc3_W,
                  fc3_b[None, :])
    return pred.reshape(-1)


# speedup vs baseline: 1.1327x; 1.1327x over previous
import functools

import jax
import jax.numpy as jnp
from jax import lax
from jax.experimental import pallas as pl
from jax.experimental.pallas import tpu as pltpu
from jax.experimental.pallas import tpu_sc as plsc

N = 10000
M = 32
NORIG = 92
A = 128
BF = 16
C2 = 256
EPS = 1e-5
PAIRS = N * M

NB = 80
GRID_P = N // NB
RB = 1000
GRID_N = N // RB
CHUNK = 80
NW = 32
BPW = PAIRS // NW
NCHUNK = BPW // CHUNK


def _softplus(x):
    return jnp.maximum(x, 0.0) + jnp.log1p(jnp.exp(-jnp.abs(x)))


def _sigmoid(x):
    return 1.0 / (1.0 + jnp.exp(-x))


def _embed_body(x_ref, w0_ref, b0_ref, w1_ref, bfc_ref, emb_ref, p1_ref):
    x = x_ref[...]
    emb = jnp.dot(x, w0_ref[...], preferred_element_type=jnp.float32, precision=lax.Precision.HIGHEST) + b0_ref[...]
    emb_ref[...] = emb
    p1_ref[...] = (
        jnp.dot(emb, w1_ref[...], preferred_element_type=jnp.float32, precision=lax.Precision.HIGHEST) + bfc_ref[...]
    )


def _embed(xpad, w0pad, b0, w1, bfc):
    return pl.pallas_call(
        _embed_body,
        grid=(GRID_N,),
        in_specs=[
            pl.BlockSpec((RB, A), lambda i: (i, 0)),
            pl.BlockSpec((A, A), lambda i: (0, 0)),
            pl.BlockSpec((1, A), lambda i: (0, 0)),
            pl.BlockSpec((A, C2), lambda i: (0, 0)),
            pl.BlockSpec((1, C2), lambda i: (0, 0)),
        ],
        out_specs=[
            pl.BlockSpec((RB, A), lambda i: (i, 0)),
            pl.BlockSpec((RB, C2), lambda i: (i, 0)),
        ],
        out_shape=[
            jax.ShapeDtypeStruct((N, A), jnp.float32),
            jax.ShapeDtypeStruct((N, C2), jnp.float32),
        ],
    )(xpad, w0pad, b0, w1, bfc)


def _update_body(prev_ref, nbr_ref, sums2_ref, g2_ref, b2_ref, w1_ref, bfc_ref,
                 s_ref, p1_ref):
    mean = sums2_ref[0:1, :] / N
    var = sums2_ref[1:2, :] / N - mean * mean
    scale = g2_ref[...] * lax.rsqrt(var + EPS)
    shift = b2_ref[...] - mean * scale
    s = _softplus(prev_ref[...] + nbr_ref[...] * scale + shift)
    s_ref[...] = s
    p1_ref[...] = (
        jnp.dot(s, w1_ref[...], preferred_element_type=jnp.float32, precision=lax.Precision.HIGHEST) + bfc_ref[...]
    )


def _update(prev, nbr, sums2, g2, b2, w1, bfc):
    return pl.pallas_call(
        _update_body,
        grid=(GRID_N,),
        in_specs=[
            pl.BlockSpec((RB, A), lambda i: (i, 0)),
            pl.BlockSpec((RB, A), lambda i: (i, 0)),
            pl.BlockSpec((2, A), lambda i: (0, 0)),
            pl.BlockSpec((1, A), lambda i: (0, 0)),
            pl.BlockSpec((1, A), lambda i: (0, 0)),
            pl.BlockSpec((A, C2), lambda i: (0, 0)),
            pl.BlockSpec((1, C2), lambda i: (0, 0)),
        ],
        out_specs=[
            pl.BlockSpec((RB, A), lambda i: (i, 0)),
            pl.BlockSpec((RB, C2), lambda i: (i, 0)),
        ],
        out_shape=[
            jax.ShapeDtypeStruct((N, A), jnp.float32),
            jax.ShapeDtypeStruct((N, C2), jnp.float32),
        ],
    )(prev, nbr, sums2, g2, b2, w1, bfc)


def _gather_rows(table, idx_flat):
    info = plsc.get_sparse_core_info()
    nc = info.num_cores

    @functools.partial(
        pl.kernel,
        mesh=plsc.VectorSubcoreMesh(core_axis_name="c", subcore_axis_name="s"),
        out_type=jax.ShapeDtypeStruct((PAIRS, A), jnp.float32),
        scratch_types=[
            pltpu.VMEM((BPW,), jnp.int32),
            pltpu.VMEM((2, CHUNK, A), jnp.float32),
            pltpu.SemaphoreType.DMA,
            pltpu.SemaphoreType.DMA,
        ],
    )
    def k(table_hbm, idx_hbm, out_hbm, idx_v, rows_v, sem0, sem1):
        wid = lax.axis_index("s") * nc + lax.axis_index("c")
        base = pl.multiple_of(wid * BPW, 8)
        pltpu.sync_copy(idx_hbm.at[pl.ds(base, BPW)], idx_v)
        sems = (sem0, sem1)
        cps = [None, None]
        for b in range(2):
            cps[b] = pltpu.async_copy(
                table_hbm.at[idx_v.at[pl.ds(b * CHUNK, CHUNK)]],
                rows_v.at[b], sems[b])
        for kk in range(NCHUNK):
            b = kk % 2
            cps[b].wait()
            pltpu.sync_copy(rows_v.at[b],
                            out_hbm.at[pl.ds(base + kk * CHUNK, CHUNK)])
            nxt = kk + 2
            if nxt < NCHUNK:
                cps[b] = pltpu.async_copy(
                    table_hbm.at[idx_v.at[pl.ds(nxt * CHUNK, CHUNK)]],
                    rows_v.at[b], sems[b])

    return k(table, idx_flat)


def _stats_body(g_ref, bond_ref, p1_ref, w2_ref, w3_ref, out_ref,
                gg_ref, cgb_ref, cbb_ref, v_ref, sg_ref, sb_ref):
    i = pl.program_id(0)
    hp = dict(preferred_element_type=jnp.float32,
              precision=lax.Precision.HIGHEST)
    tdim = (((0,), (0,)), ((), ()))

    @pl.when(i == 0)
    def _init():
        gg_ref[...] = jnp.zeros_like(gg_ref)
        cgb_ref[...] = jnp.zeros_like(cgb_ref)
        cbb_ref[...] = jnp.zeros_like(cbb_ref)
        v_ref[...] = jnp.zeros_like(v_ref)
        sg_ref[...] = jnp.zeros_like(sg_ref)
        sb_ref[...] = jnp.zeros_like(sb_ref)

    g3 = g_ref[...]
    b3 = bond_ref[...]
    p1 = p1_ref[...]
    w2 = w2_ref[...]
    w3 = w3_ref[...]
    g2 = g3.reshape(NB * M, A)
    b2 = b3.reshape(NB * M, BF)
    gg_ref[...] += lax.dot_general(g2, g2, tdim, **hp)
    cgb_ref[...] += lax.dot_general(g2, b2, tdim, **hp)
    cbb_ref[...] += lax.dot_general(b2, b2, tdim, **hp)
    gs = jnp.sum(g3, axis=1)
    bs = jnp.sum(b3, axis=1)
    v_ref[0:1, :] += jnp.sum(p1, axis=0, keepdims=True)
    v_ref[1:2, :] += jnp.sum(p1 * p1, axis=0, keepdims=True)
    v_ref[2:3, :] += jnp.sum(p1 * jnp.dot(gs, w2, **hp), axis=0, keepdims=True)
    v_ref[3:4, :] += jnp.sum(p1 * jnp.dot(bs, w3, **hp), axis=0, keepdims=True)
    sg_ref[...] += jnp.sum(gs, axis=0, keepdims=True)
    sb_ref[...] += jnp.sum(bs, axis=0, keepdims=True)

    @pl.when(i == GRID_P - 1)
    def _fin():
        sum_t = (M * v_ref[0:1, :] + jnp.dot(sg_ref[...], w2, **hp)
                 + jnp.dot(sb_ref[...], w3, **hp))
        q2 = jnp.sum(w2 * jnp.dot(gg_ref[...], w2, **hp), axis=0, keepdims=True)
        r2 = jnp.sum(w3 * jnp.dot(cbb_ref[...], w3, **hp), axis=0, keepdims=True)
        qr = jnp.sum(w2 * jnp.dot(cgb_ref[...], w3, **hp), axis=0, keepdims=True)
        sumsq_t = (M * v_ref[1:2, :] + q2 + r2
                   + 2.0 * (v_ref[2:3, :] + v_ref[3:4, :] + qr))
        out_ref[0:1, :] = sum_t
        out_ref[1:2, :] = sumsq_t


def _stats(g3, bond3, p1, w2, w3):
    return pl.pallas_call(
        _stats_body,
        grid=(GRID_P,),
        in_specs=[
            pl.BlockSpec((NB, M, A), lambda i: (i, 0, 0)),
            pl.BlockSpec((NB, M, BF), lambda i: (i, 0, 0)),
            pl.BlockSpec((NB, C2), lambda i: (i, 0)),
            pl.BlockSpec((A, C2), lambda i: (0, 0)),
            pl.BlockSpec((BF, C2), lambda i: (0, 0)),
        ],
        out_specs=pl.BlockSpec((2, C2), lambda i: (0, 0)),
        out_shape=jax.ShapeDtypeStruct((2, C2), jnp.float32),
        scratch_shapes=[
            pltpu.VMEM((A, A), jnp.float32),
            pltpu.VMEM((A, BF), jnp.float32),
            pltpu.VMEM((BF, BF), jnp.float32),
            pltpu.VMEM((4, C2), jnp.float32),
            pltpu.VMEM((1, A), jnp.float32),
            pltpu.VMEM((1, BF), jnp.float32),
        ],
    )(g3, bond3, p1, w2, w3)


def _main_body(g_ref, bond_ref, p1_ref, w2_ref, w3_ref, sums_ref, g1_ref, b1_ref,
               nbr_ref, out2_ref, acc_ref):
    i = pl.program_id(0)

    @pl.when(i == 0)
    def _init():
        acc_ref[...] = jnp.zeros_like(acc_ref)

    mean = sums_ref[0:1, :] / PAIRS
    var = sums_ref[1:2, :] / PAIRS - mean * mean
    scale = g1_ref[...] * lax.rsqrt(var + EPS)
    shift = b1_ref[...] - mean * scale

    g = g_ref[...].reshape(NB * M, A)
    bond = bond_ref[...].reshape(NB * M, BF)
    t = (jnp.dot(g, w2_ref[...], preferred_element_type=jnp.float32, precision=lax.Precision.HIGHEST)
         + jnp.dot(bond, w3_ref[...], preferred_element_type=jnp.float32, precision=lax.Precision.HIGHEST))
    t = (t.reshape(NB, M, C2) + p1_ref[...][:, None, :]).reshape(NB * M, C2)
    t = t * scale + shift
    f = _sigmoid(t[:, :A])
    c = _softplus(t[:, A:])
    nbr = jnp.sum((f * c).reshape(NB, M, A), axis=1)
    nbr_ref[...] = nbr
    acc_ref[0:1, :] += jnp.sum(nbr, axis=0, keepdims=True)
    acc_ref[1:2, :] += jnp.sum(nbr * nbr, axis=0, keepdims=True)

    @pl.when(i == GRID_P - 1)
    def _fin():
        out2_ref[...] = acc_ref[...]


def _main(g3, bond3, p1, w2, w3, sums, g1, b1):
    return pl.pallas_call(
        _main_body,
        grid=(GRID_P,),
        in_specs=[
            pl.BlockSpec((NB, M, A), lambda i: (i, 0, 0)),
            pl.BlockSpec((NB, M, BF), lambda i: (i, 0, 0)),
            pl.BlockSpec((NB, C2), lambda i: (i, 0)),
            pl.BlockSpec((A, C2), lambda i: (0, 0)),
            pl.BlockSpec((BF, C2), lambda i: (0, 0)),
            pl.BlockSpec((2, C2), lambda i: (0, 0)),
            pl.BlockSpec((1, C2), lambda i: (0, 0)),
            pl.BlockSpec((1, C2), lambda i: (0, 0)),
        ],
        out_specs=[
            pl.BlockSpec((NB, A), lambda i: (i, 0)),
            pl.BlockSpec((2, A), lambda i: (0, 0)),
        ],
        out_shape=[
            jax.ShapeDtypeStruct((N, A), jnp.float32),
            jax.ShapeDtypeStruct((2, A), jnp.float32),
        ],
        scratch_shapes=[pltpu.VMEM((2, A), jnp.float32)],
    )(g3, bond3, p1, w2, w3, sums, g1, b1)


def _final_body(prev_ref, nbr_ref, sums2_ref, g2_ref, b2_ref, fap_ref,
                w2a_ref, w2b_ref, b2f_ref, w3f_ref, b3f_ref, out_ref, acc_ref):
    i = pl.program_id(0)

    @pl.when(i == 0)
    def _init():
        acc_ref[...] = jnp.zeros_like(acc_ref)

    mean = sums2_ref[0:1, :] / N
    var = sums2_ref[1:2, :] / N - mean * mean
    scale = g2_ref[...] * lax.rsqrt(var + EPS)
    shift = b2_ref[...] - mean * scale
    s = _softplus(prev_ref[...] + nbr_ref[...] * scale + shift)
    acc_ref[...] += jnp.sum(s, axis=0, keepdims=True)

    @pl.when(i == GRID_N - 1)
    def _fin():
        feat = acc_ref[...] / N
        h = (jnp.dot(_softplus(feat), w2a_ref[...],
                     preferred_element_type=jnp.float32, precision=lax.Precision.HIGHEST)
             + _softplus(fap_ref[...]) * w2b_ref[...] + b2f_ref[...])
        out_ref[...] = (jnp.dot(_softplus(h), w3f_ref[...],
                                preferred_element_type=jnp.float32, precision=lax.Precision.HIGHEST)
                        + b3f_ref[...])


def _final(prev, nbr, sums2, g2, b2, fap, w2a, w2b, b2f, w3f, b3f):
    return pl.pallas_call(
        _final_body,
        grid=(GRID_N,),
        in_specs=[
            pl.BlockSpec((RB, A), lambda i: (i, 0)),
            pl.BlockSpec((RB, A), lambda i: (i, 0)),
            pl.BlockSpec((2, A), lambda i: (0, 0)),
            pl.BlockSpec((1, A), lambda i: (0, 0)),
            pl.BlockSpec((1, A), lambda i: (0, 0)),
            pl.BlockSpec((1, 1), lambda i: (0, 0)),
            pl.BlockSpec((A, HID), lambda i: (0, 0)),
            pl.BlockSpec((1, HID), lambda i: (0, 0)),
            pl.BlockSpec((1, HID), lambda i: (0, 0)),
            pl.BlockSpec((HID, 1), lambda i: (0, 0)),
            pl.BlockSpec((1, 1), lambda i: (0, 0)),
        ],
        out_specs=pl.BlockSpec((1, 1), lambda i: (0, 0)),
        out_shape=jax.ShapeDtypeStruct((1, 1), jnp.float32),
        scratch_shapes=[pltpu.VMEM((1, A), jnp.float32)],
    )(prev, nbr, sums2, g2, b2, fap, w2a, w2b, b2f, w3f, b3f)


HID = 128


def kernel(site_features, bond_features, bond_indices, feature_after_pooing,
           fc1_W, fc1_b, conv_params, fc2_W, fc2_b, fc3_W, fc3_b):
    xpad = jnp.pad(site_features, ((0, 0), (0, A - NORIG)))
    w0pad = jnp.pad(fc1_W, ((0, A - NORIG), (0, 0)))
    idx = bond_indices.reshape(-1).astype(jnp.int32)
    bond3 = bond_features

    site = None
    nbr = sums2 = None
    for l in range(3):
        Wfc, bfc, g1, b1, g2p, b2p = conv_params[l]
        w1 = Wfc[:A]
        w2 = Wfc[A:2 * A]
        w3 = Wfc[2 * A:]
        if l == 0:
            site, p1 = _embed(xpad, w0pad, fc1_b[None, :], w1, bfc[None, :])
        else:
            pWfc, pbfc, pg1, pb1, pg2, pb2 = conv_params[l - 1]
            site, p1 = _update(site, nbr, sums2, pg2[None, :], pb2[None, :],
                               w1, bfc[None, :])
        g3 = _gather_rows(site, idx).reshape(N, M, A)
        sums = _stats(g3, bond3, p1, w2, w3)
        nbr, sums2 = _main(g3, bond3, p1, w2, w3, sums,
                           g1[None, :], b1[None, :])

    _, _, _, _, g2l, b2l = conv_params[2]
    pred = _final(site, nbr, sums2, g2l[None, :], b2l[None, :],
                  feature_after_pooing[:, None],
                  fc2_W[:A], fc2_W[A:A + 1], fc2_b[None, :], fc3_W,
                  fc3_b[None, :])
    return pred.reshape(-1)


# speedup vs baseline: 1.6504x; 1.4570x over previous
import functools

import jax
import jax.numpy as jnp
from jax import lax
from jax.experimental import pallas as pl
from jax.experimental.pallas import tpu as pltpu
from jax.experimental.pallas import tpu_sc as plsc

N = 10000
M = 32
NORIG = 92
A = 128
BF = 16
C2 = 256
EPS = 1e-5
PAIRS = N * M

NB = 80
GRID_P = N // NB
RB = 1000
GRID_N = N // RB
CHUNK = 80
NW = 32
BPW = PAIRS // NW
NCHUNK = BPW // CHUNK


def _softplus(x):
    return jnp.maximum(x, 0.0) + jnp.log1p(jnp.exp(-jnp.abs(x)))


def _split_bf16(x):
    hi = x.astype(jnp.bfloat16)
    lo = (x - hi.astype(jnp.float32)).astype(jnp.bfloat16)
    return hi, lo


def _dot3(xh, xl, wh, wl, dims=None):
    if dims is None:
        mm = lambda a, b: jnp.dot(a, b, preferred_element_type=jnp.float32)
    else:
        mm = lambda a, b: lax.dot_general(a, b, dims,
                                          preferred_element_type=jnp.float32)
    return mm(xh, wh) + mm(xh, wl) + mm(xl, wh)


def _sigmoid(x):
    return 1.0 / (1.0 + jnp.exp(-x))


def _embed_body(x_ref, w0_ref, b0_ref, w1_ref, bfc_ref, emb_ref, p1_ref):
    x = x_ref[...]
    emb = jnp.dot(x, w0_ref[...], preferred_element_type=jnp.float32, precision=lax.Precision.HIGHEST) + b0_ref[...]
    emb_ref[...] = emb
    p1_ref[...] = (
        jnp.dot(emb, w1_ref[...], preferred_element_type=jnp.float32, precision=lax.Precision.HIGHEST) + bfc_ref[...]
    )


def _embed(xpad, w0pad, b0, w1, bfc):
    return pl.pallas_call(
        _embed_body,
        grid=(GRID_N,),
        in_specs=[
            pl.BlockSpec((RB, A), lambda i: (i, 0)),
            pl.BlockSpec((A, A), lambda i: (0, 0)),
            pl.BlockSpec((1, A), lambda i: (0, 0)),
            pl.BlockSpec((A, C2), lambda i: (0, 0)),
            pl.BlockSpec((1, C2), lambda i: (0, 0)),
        ],
        out_specs=[
            pl.BlockSpec((RB, A), lambda i: (i, 0)),
            pl.BlockSpec((RB, C2), lambda i: (i, 0)),
        ],
        out_shape=[
            jax.ShapeDtypeStruct((N, A), jnp.float32),
            jax.ShapeDtypeStruct((N, C2), jnp.float32),
        ],
    )(xpad, w0pad, b0, w1, bfc)


def _update_body(prev_ref, nbr_ref, sums2_ref, g2_ref, b2_ref, w1_ref, bfc_ref,
                 s_ref, p1_ref):
    mean = sums2_ref[0:1, :] / N
    var = sums2_ref[1:2, :] / N - mean * mean
    scale = g2_ref[...] * lax.rsqrt(var + EPS)
    shift = b2_ref[...] - mean * scale
    s = _softplus(prev_ref[...] + nbr_ref[...] * scale + shift)
    s_ref[...] = s
    p1_ref[...] = (
        jnp.dot(s, w1_ref[...], preferred_element_type=jnp.float32, precision=lax.Precision.HIGHEST) + bfc_ref[...]
    )


def _update(prev, nbr, sums2, g2, b2, w1, bfc):
    return pl.pallas_call(
        _update_body,
        grid=(GRID_N,),
        in_specs=[
            pl.BlockSpec((RB, A), lambda i: (i, 0)),
            pl.BlockSpec((RB, A), lambda i: (i, 0)),
            pl.BlockSpec((2, A), lambda i: (0, 0)),
            pl.BlockSpec((1, A), lambda i: (0, 0)),
            pl.BlockSpec((1, A), lambda i: (0, 0)),
            pl.BlockSpec((A, C2), lambda i: (0, 0)),
            pl.BlockSpec((1, C2), lambda i: (0, 0)),
        ],
        out_specs=[
            pl.BlockSpec((RB, A), lambda i: (i, 0)),
            pl.BlockSpec((RB, C2), lambda i: (i, 0)),
        ],
        out_shape=[
            jax.ShapeDtypeStruct((N, A), jnp.float32),
            jax.ShapeDtypeStruct((N, C2), jnp.float32),
        ],
    )(prev, nbr, sums2, g2, b2, w1, bfc)


def _gather_rows(table, idx_flat):
    info = plsc.get_sparse_core_info()
    nc = info.num_cores

    @functools.partial(
        pl.kernel,
        mesh=plsc.VectorSubcoreMesh(core_axis_name="c", subcore_axis_name="s"),
        out_type=jax.ShapeDtypeStruct((PAIRS, A), jnp.float32),
        scratch_types=[
            pltpu.VMEM((BPW,), jnp.int32),
            pltpu.VMEM((2, CHUNK, A), jnp.float32),
            pltpu.SemaphoreType.DMA,
            pltpu.SemaphoreType.DMA,
        ],
    )
    def k(table_hbm, idx_hbm, out_hbm, idx_v, rows_v, sem0, sem1):
        wid = lax.axis_index("s") * nc + lax.axis_index("c")
        base = pl.multiple_of(wid * BPW, 8)
        pltpu.sync_copy(idx_hbm.at[pl.ds(base, BPW)], idx_v)
        sems = (sem0, sem1)
        cps = [None, None]
        for b in range(2):
            cps[b] = pltpu.async_copy(
                table_hbm.at[idx_v.at[pl.ds(b * CHUNK, CHUNK)]],
                rows_v.at[b], sems[b])
        for kk in range(NCHUNK):
            b = kk % 2
            cps[b].wait()
            pltpu.sync_copy(rows_v.at[b],
                            out_hbm.at[pl.ds(base + kk * CHUNK, CHUNK)])
            nxt = kk + 2
            if nxt < NCHUNK:
                cps[b] = pltpu.async_copy(
                    table_hbm.at[idx_v.at[pl.ds(nxt * CHUNK, CHUNK)]],
                    rows_v.at[b], sems[b])

    return k(table, idx_flat)


def _stats_body(g_ref, bond_ref, p1_ref, w2_ref, w3_ref, out_ref,
                gg_ref, cgb_ref, cbb_ref, v_ref, sg_ref, sb_ref):
    i = pl.program_id(0)
    hp = dict(preferred_element_type=jnp.float32,
              precision=lax.Precision.HIGHEST)
    tdim = (((0,), (0,)), ((), ()))

    @pl.when(i == 0)
    def _init():
        gg_ref[...] = jnp.zeros_like(gg_ref)
        cgb_ref[...] = jnp.zeros_like(cgb_ref)
        cbb_ref[...] = jnp.zeros_like(cbb_ref)
        v_ref[...] = jnp.zeros_like(v_ref)
        sg_ref[...] = jnp.zeros_like(sg_ref)
        sb_ref[...] = jnp.zeros_like(sb_ref)

    g3 = g_ref[...]
    b3 = bond_ref[...]
    p1 = p1_ref[...]
    w2 = w2_ref[...]
    w3 = w3_ref[...]
    g2 = g3.reshape(NB * M, A)
    b2 = b3.reshape(NB * M, BF)
    gh, gl = _split_bf16(g2)
    bh, bl = _split_bf16(b2)
    gg_ref[...] += _dot3(gh, gl, gh, gl, tdim)
    cgb_ref[...] += _dot3(gh, gl, bh, bl, tdim)
    cbb_ref[...] += _dot3(bh, bl, bh, bl, tdim)
    gs = jnp.sum(g3, axis=1)
    bs = jnp.sum(b3, axis=1)
    v_ref[0:1, :] += jnp.sum(p1, axis=0, keepdims=True)
    v_ref[1:2, :] += jnp.sum(p1 * p1, axis=0, keepdims=True)
    v_ref[2:3, :] += jnp.sum(p1 * jnp.dot(gs, w2, **hp), axis=0, keepdims=True)
    v_ref[3:4, :] += jnp.sum(p1 * jnp.dot(bs, w3, **hp), axis=0, keepdims=True)
    sg_ref[...] += jnp.sum(gs, axis=0, keepdims=True)
    sb_ref[...] += jnp.sum(bs, axis=0, keepdims=True)

    @pl.when(i == GRID_P - 1)
    def _fin():
        sum_t = (M * v_ref[0:1, :] + jnp.dot(sg_ref[...], w2, **hp)
                 + jnp.dot(sb_ref[...], w3, **hp))
        q2 = jnp.sum(w2 * jnp.dot(gg_ref[...], w2, **hp), axis=0, keepdims=True)
        r2 = jnp.sum(w3 * jnp.dot(cbb_ref[...], w3, **hp), axis=0, keepdims=True)
        qr = jnp.sum(w2 * jnp.dot(cgb_ref[...], w3, **hp), axis=0, keepdims=True)
        sumsq_t = (M * v_ref[1:2, :] + q2 + r2
                   + 2.0 * (v_ref[2:3, :] + v_ref[3:4, :] + qr))
        out_ref[0:1, :] = sum_t
        out_ref[1:2, :] = sumsq_t


def _stats(g3, bond3, p1, w2, w3):
    return pl.pallas_call(
        _stats_body,
        grid=(GRID_P,),
        in_specs=[
            pl.BlockSpec((NB, M, A), lambda i: (i, 0, 0)),
            pl.BlockSpec((NB, M, BF), lambda i: (i, 0, 0)),
            pl.BlockSpec((NB, C2), lambda i: (i, 0)),
            pl.BlockSpec((A, C2), lambda i: (0, 0)),
            pl.BlockSpec((BF, C2), lambda i: (0, 0)),
        ],
        out_specs=pl.BlockSpec((2, C2), lambda i: (0, 0)),
        out_shape=jax.ShapeDtypeStruct((2, C2), jnp.float32),
        scratch_shapes=[
            pltpu.VMEM((A, A), jnp.float32),
            pltpu.VMEM((A, BF), jnp.float32),
            pltpu.VMEM((BF, BF), jnp.float32),
            pltpu.VMEM((4, C2), jnp.float32),
            pltpu.VMEM((1, A), jnp.float32),
            pltpu.VMEM((1, BF), jnp.float32),
        ],
    )(g3, bond3, p1, w2, w3)


def _main_body(g_ref, bond_ref, p1_ref, w2h_ref, w2l_ref, w3h_ref, w3l_ref,
               sums_ref, g1_ref, b1_ref, nbr_ref, out2_ref, acc_ref):
    i = pl.program_id(0)

    @pl.when(i == 0)
    def _init():
        acc_ref[...] = jnp.zeros_like(acc_ref)

    mean = sums_ref[0:1, :] / PAIRS
    var = sums_ref[1:2, :] / PAIRS - mean * mean
    scale = g1_ref[...] * lax.rsqrt(var + EPS)
    shift = b1_ref[...] - mean * scale

    g = g_ref[...].reshape(NB * M, A)
    bond = bond_ref[...].reshape(NB * M, BF)
    gh, gl = _split_bf16(g)
    bh, bl = _split_bf16(bond)
    t = (_dot3(gh, gl, w2h_ref[...], w2l_ref[...])
         + _dot3(bh, bl, w3h_ref[...], w3l_ref[...]))
    t = (t.reshape(NB, M, C2) + p1_ref[...][:, None, :]).reshape(NB * M, C2)
    t = t * scale + shift
    e = jnp.exp(-jnp.abs(t))
    eh = e[:, :A]
    r = 1.0 / (1.0 + eh)
    f = jnp.where(t[:, :A] >= 0, r, eh * r)
    c = jnp.maximum(t[:, A:], 0.0) + jnp.log1p(e[:, A:])
    nbr = jnp.sum((f * c).reshape(NB, M, A), axis=1)
    nbr_ref[...] = nbr
    acc_ref[0:1, :] += jnp.sum(nbr, axis=0, keepdims=True)
    acc_ref[1:2, :] += jnp.sum(nbr * nbr, axis=0, keepdims=True)

    @pl.when(i == GRID_P - 1)
    def _fin():
        out2_ref[...] = acc_ref[...]


def _main(g3, bond3, p1, w2h, w2l, w3h, w3l, sums, g1, b1):
    return pl.pallas_call(
        _main_body,
        grid=(GRID_P,),
        in_specs=[
            pl.BlockSpec((NB, M, A), lambda i: (i, 0, 0)),
            pl.BlockSpec((NB, M, BF), lambda i: (i, 0, 0)),
            pl.BlockSpec((NB, C2), lambda i: (i, 0)),
            pl.BlockSpec((A, C2), lambda i: (0, 0)),
            pl.BlockSpec((A, C2), lambda i: (0, 0)),
            pl.BlockSpec((BF, C2), lambda i: (0, 0)),
            pl.BlockSpec((BF, C2), lambda i: (0, 0)),
            pl.BlockSpec((2, C2), lambda i: (0, 0)),
            pl.BlockSpec((1, C2), lambda i: (0, 0)),
            pl.BlockSpec((1, C2), lambda i: (0, 0)),
        ],
        out_specs=[
            pl.BlockSpec((NB, A), lambda i: (i, 0)),
            pl.BlockSpec((2, A), lambda i: (0, 0)),
        ],
        out_shape=[
            jax.ShapeDtypeStruct((N, A), jnp.float32),
            jax.ShapeDtypeStruct((2, A), jnp.float32),
        ],
        scratch_shapes=[pltpu.VMEM((2, A), jnp.float32)],
    )(g3, bond3, p1, w2h, w2l, w3h, w3l, sums, g1, b1)


def _final_body(prev_ref, nbr_ref, sums2_ref, g2_ref, b2_ref, fap_ref,
                w2a_ref, w2b_ref, b2f_ref, w3f_ref, b3f_ref, out_ref, acc_ref):
    i = pl.program_id(0)

    @pl.when(i == 0)
    def _init():
        acc_ref[...] = jnp.zeros_like(acc_ref)

    mean = sums2_ref[0:1, :] / N
    var = sums2_ref[1:2, :] / N - mean * mean
    scale = g2_ref[...] * lax.rsqrt(var + EPS)
    shift = b2_ref[...] - mean * scale
    s = _softplus(prev_ref[...] + nbr_ref[...] * scale + shift)
    acc_ref[...] += jnp.sum(s, axis=0, keepdims=True)

    @pl.when(i == GRID_N - 1)
    def _fin():
        feat = acc_ref[...] / N
        h = (jnp.dot(_softplus(feat), w2a_ref[...],
                     preferred_element_type=jnp.float32, precision=lax.Precision.HIGHEST)
             + _softplus(fap_ref[...]) * w2b_ref[...] + b2f_ref[...])
        out_ref[...] = (jnp.dot(_softplus(h), w3f_ref[...],
                                preferred_element_type=jnp.float32, precision=lax.Precision.HIGHEST)
                        + b3f_ref[...])


def _final(prev, nbr, sums2, g2, b2, fap, w2a, w2b, b2f, w3f, b3f):
    return pl.pallas_call(
        _final_body,
        grid=(GRID_N,),
        in_specs=[
            pl.BlockSpec((RB, A), lambda i: (i, 0)),
            pl.BlockSpec((RB, A), lambda i: (i, 0)),
            pl.BlockSpec((2, A), lambda i: (0, 0)),
            pl.BlockSpec((1, A), lambda i: (0, 0)),
            pl.BlockSpec((1, A), lambda i: (0, 0)),
            pl.BlockSpec((1, 1), lambda i: (0, 0)),
            pl.BlockSpec((A, HID), lambda i: (0, 0)),
            pl.BlockSpec((1, HID), lambda i: (0, 0)),
            pl.BlockSpec((1, HID), lambda i: (0, 0)),
            pl.BlockSpec((HID, 1), lambda i: (0, 0)),
            pl.BlockSpec((1, 1), lambda i: (0, 0)),
        ],
        out_specs=pl.BlockSpec((1, 1), lambda i: (0, 0)),
        out_shape=jax.ShapeDtypeStruct((1, 1), jnp.float32),
        scratch_shapes=[pltpu.VMEM((1, A), jnp.float32)],
    )(prev, nbr, sums2, g2, b2, fap, w2a, w2b, b2f, w3f, b3f)


HID = 128


def kernel(site_features, bond_features, bond_indices, feature_after_pooing,
           fc1_W, fc1_b, conv_params, fc2_W, fc2_b, fc3_W, fc3_b):
    xpad = jnp.pad(site_features, ((0, 0), (0, A - NORIG)))
    w0pad = jnp.pad(fc1_W, ((0, A - NORIG), (0, 0)))
    idx = bond_indices.reshape(-1).astype(jnp.int32)
    bond3 = bond_features

    site = None
    nbr = sums2 = None
    for l in range(3):
        Wfc, bfc, g1, b1, g2p, b2p = conv_params[l]
        w1 = Wfc[:A]
        w2 = Wfc[A:2 * A]
        w3 = Wfc[2 * A:]
        if l == 0:
            site, p1 = _embed(xpad, w0pad, fc1_b[None, :], w1, bfc[None, :])
        else:
            pWfc, pbfc, pg1, pb1, pg2, pb2 = conv_params[l - 1]
            site, p1 = _update(site, nbr, sums2, pg2[None, :], pb2[None, :],
                               w1, bfc[None, :])
        w2h = w2.astype(jnp.bfloat16)
        w2l = (w2 - w2h.astype(jnp.float32)).astype(jnp.bfloat16)
        w3h = w3.astype(jnp.bfloat16)
        w3l = (w3 - w3h.astype(jnp.float32)).astype(jnp.bfloat16)
        g3 = _gather_rows(site, idx).reshape(N, M, A)
        sums = _stats(g3, bond3, p1, w2, w3)
        nbr, sums2 = _main(g3, bond3, p1, w2h, w2l, w3h, w3l, sums,
                           g1[None, :], b1[None, :])

    _, _, _, _, g2l, b2l = conv_params[2]
    pred = _final(site, nbr, sums2, g2l[None, :], b2l[None, :],
                  feature_after_pooing[:, None],
                  fc2_W[:A], fc2_W[A:A + 1], fc2_b[None, :], fc3_W,
                  fc3_b[None, :])
    return pred.reshape(-1)
